# baseline (speedup 1.0000x reference)
import functools

import jax
import jax.numpy as jnp
import numpy as np
from jax import lax
from jax.experimental import pallas as pl
from jax.experimental.pallas import tpu as pltpu
from jax.experimental.pallas import tpu_sc as plsc

_FIELD_DIMS = [38462] * 26
_OFFSETS = np.concatenate([[0], np.cumsum(_FIELD_DIMS)[:-1]]).astype(np.int32)
_VOCAB = int(np.sum(_FIELD_DIMS))
_ED = 16
_B = 16384
_NF = 26
_HALF = _B // 2
_NIDXH = _HALF * _NF
_V16 = (_VOCAB + 15) // 16

_NC = 2
_NS = 16
_NW = _NC * _NS
_B_PER_W = _NIDXH // _NW
_SC_CHUNK = 832
_N_SC_CHUNK = _B_PER_W // _SC_CHUNK

_TC_CHUNK = 1024
_N_TC_CHUNK = _HALF // _TC_CHUNK
_D1 = _NF * _ED


def _sc_gather(embed2, e1pad, idx_flat, idx_hi, idx_lo):
  mesh = plsc.VectorSubcoreMesh(core_axis_name="c", subcore_axis_name="s")

  @functools.partial(
      pl.kernel,
      mesh=mesh,
      compiler_params=pltpu.CompilerParams(
          use_tc_tiling_on_sc=False, needs_layout_passes=False),
      out_type=[
          jax.ShapeDtypeStruct((_NIDXH, _ED), jnp.float32),
          jax.ShapeDtypeStruct((_NIDXH,), jnp.float32),
      ],
      scratch_types=[
          pltpu.VMEM((_SC_CHUNK,), jnp.int32),
          pltpu.VMEM((_SC_CHUNK,), jnp.int32),
          pltpu.VMEM((_SC_CHUNK,), jnp.int32),
          pltpu.VMEM((_SC_CHUNK, _ED), jnp.float32),
          pltpu.VMEM((_SC_CHUNK, _ED), jnp.float32),
          pltpu.VMEM((_SC_CHUNK,), jnp.float32),
          pltpu.SemaphoreType.DMA,
          pltpu.SemaphoreType.DMA,
      ],
  )
  def k(e2r, e1r, idx_hbm, hi_hbm, lo_hbm, v_hbm, w_hbm,
        idx_v, hi_v, lo_v, rows_v, e1rows_v, w_v, sem2, sem1):
    wid = lax.axis_index("s") * _NC + lax.axis_index("c")
    base = wid * _B_PER_W

    @pl.loop(0, _N_SC_CHUNK)
    def _(c):
      off = base + c * _SC_CHUNK
      pltpu.sync_copy(idx_hbm.at[pl.ds(off, _SC_CHUNK)], idx_v)
      pltpu.sync_copy(hi_hbm.at[pl.ds(off, _SC_CHUNK)], hi_v)
      pltpu.sync_copy(lo_hbm.at[pl.ds(off, _SC_CHUNK)], lo_v)
      cp2 = pltpu.make_async_copy(e2r.at[idx_v], rows_v, sem2)
      cp2.start()
      cp1 = pltpu.make_async_copy(e1r.at[hi_v], e1rows_v, sem1)
      cp1.start()
      cp2.wait()
      cp1.wait()
      pltpu.sync_copy(rows_v, v_hbm.at[pl.ds(off, _SC_CHUNK)])

      rowi = lax.iota(jnp.int32, 16)

      @pl.loop(0, _SC_CHUNK // 16)
      def _(j):
        lanes = lo_v[pl.ds(j * 16, 16)]
        w_v[pl.ds(j * 16, 16)] = plsc.load_gather(
            e1rows_v, [rowi + j * 16, lanes])

      pltpu.sync_copy(w_v, w_hbm.at[pl.ds(off, _SC_CHUNK)])

  return k(embed2, e1pad, idx_flat, idx_hi, idx_lo)


def _h1_body(v_ref, w_ref, w1v_ref, w1w_ref, b1_ref, h1_ref):
  h1_ref[...] = (
      jnp.dot(w_ref[...], w1w_ref[...], preferred_element_type=jnp.float32)
      + jnp.dot(v_ref[...], w1v_ref[...], preferred_element_type=jnp.float32)
      + b1_ref[...])


def _h1(v, w, w1v, w1w, b1):
  full = lambda shape: pl.BlockSpec(shape, lambda i: (0, 0))
  return pl.pallas_call(
      _h1_body,
      grid=(_N_TC_CHUNK,),
      in_specs=[
          pl.BlockSpec((_TC_CHUNK, _D1), lambda i: (i, 0)),
          pl.BlockSpec((_TC_CHUNK, _NF), lambda i: (i, 0)),
          full(w1v.shape), full(w1w.shape), full(b1.shape),
      ],
      out_specs=pl.BlockSpec((_TC_CHUNK, 128), lambda i: (i, 0)),
      out_shape=jax.ShapeDtypeStruct((_HALF, 128), jnp.float32),
  )(v, w, w1v, w1w, b1)


def _tail_body(h1a_ref, h1b_ref, g1_ref, be1_ref, w2_ref, b2_ref, g2_ref,
               be2_ref, w3_ref, b3_ref, g3_ref, be3_ref, w4_ref, b4_ref,
               out_ref):
  def mm(a, b):
    return jnp.dot(a, b, preferred_element_type=jnp.float32)

  def bn_relu(h, g, be):
    mu = jnp.mean(h, axis=0, keepdims=True)
    var = jnp.mean((h - mu) ** 2, axis=0, keepdims=True)
    return jnp.maximum((h - mu) * lax.rsqrt(var + 1e-5) * g + be, 0.0)

  h1 = jnp.concatenate([h1a_ref[...], h1b_ref[...]], axis=0)
  h = bn_relu(h1, g1_ref[...], be1_ref[...])
  h = bn_relu(mm(h, w2_ref[...]) + b2_ref[...], g2_ref[...], be2_ref[...])
  h = bn_relu(mm(h, w3_ref[...]) + b3_ref[...], g3_ref[...], be3_ref[...])
  o = mm(h, w4_ref[...]) + b4_ref[...]
  out_ref[...] = jax.nn.sigmoid(o)


def _tail(h1a, h1b, g1, be1, w2, b2, g2, be2, w3, b3, g3, be3, w4, b4):
  full = lambda shape: pl.BlockSpec(shape, lambda i: (0, 0))
  args = (h1a, h1b, g1, be1, w2, b2, g2, be2, w3, b3, g3, be3, w4, b4)
  return pl.pallas_call(
      _tail_body,
      grid=(1,),
      in_specs=[full(a.shape) for a in args],
      out_specs=pl.BlockSpec((_B, 1), lambda i: (0, 0)),
      out_shape=jax.ShapeDtypeStruct((_B, 1), jnp.float32),
  )(*args)


def kernel(x, embed1, embed2, W1, b1, g1, be1, W2, b2, g2, be2,
           W3, b3, g3, be3, W4, b4):
  idx = (x + jnp.asarray(_OFFSETS)[None, :]).reshape(_B * _NF)
  e1pad = lax.optimization_barrier(
      jnp.pad(embed1.reshape(_VOCAB),
              (0, _V16 * 16 - _VOCAB)).reshape(_V16, _ED))
  b1r = b1.reshape(1, -1)
  w1v, w1w = W1[_NF:], W1[:_NF]

  h1_halves = []
  for h in range(2):
    ih = lax.dynamic_slice_in_dim(idx, h * _NIDXH, _NIDXH)
    v_flat, w_flat = _sc_gather(embed2, e1pad, ih, ih >> 4, ih & 15)
    h1_halves.append(_h1(v_flat.reshape(_HALF, _D1),
                         w_flat.reshape(_HALF, _NF), w1v, w1w, b1r))

  out = _tail(
      h1_halves[0], h1_halves[1],
      g1.reshape(1, -1), be1.reshape(1, -1),
      W2, b2.reshape(1, -1), g2.reshape(1, -1), be2.reshape(1, -1),
      W3, b3.reshape(1, -1), g3.reshape(1, -1), be3.reshape(1, -1),
      W4, b4.reshape(1, -1),
  )
  return out.reshape(_B)


# speedup vs baseline: 1.0154x; 1.0154x over previous
import functools

import jax
import jax.numpy as jnp
import numpy as np
from jax import lax
from jax.experimental import pallas as pl
from jax.experimental.pallas import tpu as pltpu
from jax.experimental.pallas import tpu_sc as plsc

_FIELD_DIMS = [38462] * 26
_OFFSETS = np.concatenate([[0], np.cumsum(_FIELD_DIMS)[:-1]]).astype(np.int32)
_VOCAB = int(np.sum(_FIELD_DIMS))
_ED = 16
_B = 16384
_NF = 26
_HALF = _B // 2
_NIDXH = _HALF * _NF
_V16 = (_VOCAB + 15) // 16

_NC = 2
_NS = 16
_NW = _NC * _NS
_B_PER_W = _NIDXH // _NW
_SC_CHUNK = 832
_N_SC_CHUNK = _B_PER_W // _SC_CHUNK

_TC_CHUNK = 1024
_N_TC_CHUNK = _HALF // _TC_CHUNK
_D1 = _NF * _ED


def _sc_gather(embed2, e1pad, idx_flat, idx_hi, idx_lo):
  mesh = plsc.VectorSubcoreMesh(core_axis_name="c", subcore_axis_name="s")

  @functools.partial(
      pl.kernel,
      mesh=mesh,
      compiler_params=pltpu.CompilerParams(
          use_tc_tiling_on_sc=False, needs_layout_passes=False),
      out_type=[
          jax.ShapeDtypeStruct((_NIDXH, _ED), jnp.float32),
          jax.ShapeDtypeStruct((_NIDXH,), jnp.float32),
      ],
      scratch_types=[
          pltpu.VMEM((_SC_CHUNK,), jnp.int32),
          pltpu.VMEM((_SC_CHUNK,), jnp.int32),
          pltpu.VMEM((_SC_CHUNK,), jnp.int32),
          pltpu.VMEM((_SC_CHUNK, _ED), jnp.float32),
          pltpu.VMEM((_SC_CHUNK, _ED), jnp.float32),
          pltpu.VMEM((_SC_CHUNK,), jnp.float32),
          pltpu.SemaphoreType.DMA,
          pltpu.SemaphoreType.DMA,
      ],
  )
  def k(e2r, e1r, idx_hbm, hi_hbm, lo_hbm, v_hbm, w_hbm,
        idx_v, hi_v, lo_v, rows_v, e1rows_v, w_v, sem2, sem1):
    wid = lax.axis_index("s") * _NC + lax.axis_index("c")
    base = wid * _B_PER_W

    @pl.loop(0, _N_SC_CHUNK)
    def _(c):
      off = base + c * _SC_CHUNK
      pltpu.sync_copy(idx_hbm.at[pl.ds(off, _SC_CHUNK)], idx_v)
      pltpu.sync_copy(hi_hbm.at[pl.ds(off, _SC_CHUNK)], hi_v)
      pltpu.sync_copy(lo_hbm.at[pl.ds(off, _SC_CHUNK)], lo_v)
      cp2 = pltpu.make_async_copy(e2r.at[idx_v], rows_v, sem2)
      cp2.start()
      cp1 = pltpu.make_async_copy(e1r.at[hi_v], e1rows_v, sem1)
      cp1.start()
      cp2.wait()
      cp1.wait()
      pltpu.sync_copy(rows_v, v_hbm.at[pl.ds(off, _SC_CHUNK)])

      rowi = lax.iota(jnp.int32, 16)

      @pl.loop(0, _SC_CHUNK // 16)
      def _(j):
        lanes = lo_v[pl.ds(j * 16, 16)]
        w_v[pl.ds(j * 16, 16)] = plsc.load_gather(
            e1rows_v, [rowi + j * 16, lanes])

      pltpu.sync_copy(w_v, w_hbm.at[pl.ds(off, _SC_CHUNK)])

  return k(embed2, e1pad, idx_flat, idx_hi, idx_lo)


def _h1_body(v_ref, w_ref, w1v_ref, w1w_ref, b1_ref, h1_ref):
  h1_ref[...] = (
      jnp.dot(w_ref[...], w1w_ref[...], preferred_element_type=jnp.float32)
      + jnp.dot(v_ref[...], w1v_ref[...], preferred_element_type=jnp.float32)
      + b1_ref[...])


def _h1(v, w, w1v, w1w, b1):
  full = lambda shape: pl.BlockSpec(shape, lambda i: (0, 0))
  return pl.pallas_call(
      _h1_body,
      grid=(_N_TC_CHUNK,),
      in_specs=[
          pl.BlockSpec((_TC_CHUNK, _D1), lambda i: (i, 0)),
          pl.BlockSpec((_TC_CHUNK, _NF), lambda i: (i, 0)),
          full(w1v.shape), full(w1w.shape), full(b1.shape),
      ],
      out_specs=pl.BlockSpec((_TC_CHUNK, 128), lambda i: (i, 0)),
      out_shape=jax.ShapeDtypeStruct((_HALF, 128), jnp.float32),
  )(v, w, w1v, w1w, b1)


def _tail_body(h1a_ref, h1b_ref, g1_ref, be1_ref, w2_ref, b2_ref, g2_ref,
               be2_ref, w3_ref, b3_ref, g3_ref, be3_ref, w4_ref, b4_ref,
               out_ref):
  def mm(a, b):
    return jnp.dot(a, b, preferred_element_type=jnp.float32)

  def bn_relu(h, g, be):
    mu = jnp.mean(h, axis=0, keepdims=True)
    var = jnp.mean((h - mu) ** 2, axis=0, keepdims=True)
    return jnp.maximum((h - mu) * lax.rsqrt(var + 1e-5) * g + be, 0.0)

  h1 = jnp.concatenate([h1a_ref[...], h1b_ref[...]], axis=0)
  h = bn_relu(h1, g1_ref[...], be1_ref[...])
  h = bn_relu(mm(h, w2_ref[...]) + b2_ref[...], g2_ref[...], be2_ref[...])
  h = bn_relu(mm(h, w3_ref[...]) + b3_ref[...], g3_ref[...], be3_ref[...])
  o = mm(h, w4_ref[...]) + b4_ref[...]
  out_ref[...] = jax.nn.sigmoid(o)


def _tail(h1a, h1b, g1, be1, w2, b2, g2, be2, w3, b3, g3, be3, w4, b4):
  full = lambda shape: pl.BlockSpec(shape, lambda i: (0, 0))
  args = (h1a, h1b, g1, be1, w2, b2, g2, be2, w3, b3, g3, be3, w4, b4)
  return pl.pallas_call(
      _tail_body,
      grid=(1,),
      in_specs=[full(a.shape) for a in args],
      out_specs=pl.BlockSpec((_B, 1), lambda i: (0, 0)),
      out_shape=jax.ShapeDtypeStruct((_B, 1), jnp.float32),
  )(*args)


def kernel(x, embed1, embed2, W1, b1, g1, be1, W2, b2, g2, be2,
           W3, b3, g3, be3, W4, b4):
  idx = (x + jnp.asarray(_OFFSETS)[None, :]).reshape(_B * _NF)
  e2lin = lax.optimization_barrier(
      embed2.reshape(_VOCAB * _ED)).reshape(_VOCAB, _ED)
  e1pad = lax.optimization_barrier(
      jnp.pad(embed1.reshape(_VOCAB),
              (0, _V16 * 16 - _VOCAB))).reshape(_V16, _ED)
  b1r = b1.reshape(1, -1)
  w1v, w1w = W1[_NF:], W1[:_NF]

  h1_halves = []
  for h in range(2):
    ih = lax.dynamic_slice_in_dim(idx, h * _NIDXH, _NIDXH)
    v_flat, w_flat = _sc_gather(e2lin, e1pad, ih, ih >> 4, ih & 15)
    h1_halves.append(_h1(v_flat.reshape(_HALF, _D1),
                         w_flat.reshape(_HALF, _NF), w1v, w1w, b1r))

  out = _tail(
      h1_halves[0], h1_halves[1],
      g1.reshape(1, -1), be1.reshape(1, -1),
      W2, b2.reshape(1, -1), g2.reshape(1, -1), be2.reshape(1, -1),
      W3, b3.reshape(1, -1), g3.reshape(1, -1), be3.reshape(1, -1),
      W4, b4.reshape(1, -1),
  )
  return out.reshape(_B)


# speedup vs baseline: 1.0212x; 1.0057x over previous
import functools

import jax
import jax.numpy as jnp
import numpy as np
from jax import lax
from jax.experimental import pallas as pl
from jax.experimental.pallas import tpu as pltpu
from jax.experimental.pallas import tpu_sc as plsc

_FIELD_DIMS = [38462] * 26
_OFFSETS = np.concatenate([[0], np.cumsum(_FIELD_DIMS)[:-1]]).astype(np.int32)
_VOCAB = int(np.sum(_FIELD_DIMS))
_ED = 16
_B = 16384
_NF = 26
_NIDX = _B * _NF
_V16 = (_VOCAB + 15) // 16

_NC = 2
_NS = 16
_NW = _NC * _NS
_B_PER_W = _NIDX // _NW
_SC_CHUNK = 1024
_N_SC_CHUNK = _B_PER_W // _SC_CHUNK

_TC_CHUNK = 1024
_N_TC_CHUNK = _B // _TC_CHUNK
_D1 = _NF * _ED


def _sc_gather(embed2, e1pad, idx_flat, idx_hi, idx_lo):
  mesh = plsc.VectorSubcoreMesh(core_axis_name="c", subcore_axis_name="s")

  @functools.partial(
      pl.kernel,
      mesh=mesh,
      compiler_params=pltpu.CompilerParams(
          use_tc_tiling_on_sc=False, needs_layout_passes=False),
      out_type=[
          jax.ShapeDtypeStruct((_NIDX, _ED), jnp.float32),
          jax.ShapeDtypeStruct((_NIDX,), jnp.float32),
      ],
      scratch_types=[
          pltpu.VMEM((_SC_CHUNK,), jnp.int32),
          pltpu.VMEM((_SC_CHUNK,), jnp.int32),
          pltpu.VMEM((_SC_CHUNK,), jnp.int32),
          pltpu.VMEM((_SC_CHUNK, _ED), jnp.float32),
          pltpu.VMEM((_SC_CHUNK, _ED), jnp.float32),
          pltpu.VMEM((_SC_CHUNK,), jnp.float32),
          pltpu.SemaphoreType.DMA,
          pltpu.SemaphoreType.DMA,
      ],
  )
  def k(e2r, e1r, idx_hbm, hi_hbm, lo_hbm, v_hbm, w_hbm,
        idx_v, hi_v, lo_v, rows_v, e1rows_v, w_v, sem2, sem1):
    wid = lax.axis_index("s") * _NC + lax.axis_index("c")
    base = wid * _B_PER_W

    @pl.loop(0, _N_SC_CHUNK)
    def _(c):
      off = base + c * _SC_CHUNK
      pltpu.sync_copy(idx_hbm.at[pl.ds(off, _SC_CHUNK)], idx_v)
      pltpu.sync_copy(hi_hbm.at[pl.ds(off, _SC_CHUNK)], hi_v)
      pltpu.sync_copy(lo_hbm.at[pl.ds(off, _SC_CHUNK)], lo_v)
      cp2 = pltpu.make_async_copy(e2r.at[idx_v], rows_v, sem2)
      cp2.start()
      cp1 = pltpu.make_async_copy(e1r.at[hi_v], e1rows_v, sem1)
      cp1.start()
      cp2.wait()
      cp1.wait()
      pltpu.sync_copy(rows_v, v_hbm.at[pl.ds(off, _SC_CHUNK)])

      rowi = lax.iota(jnp.int32, 16)

      @pl.loop(0, _SC_CHUNK // 16)
      def _(j):
        lanes = lo_v[pl.ds(j * 16, 16)]
        w_v[pl.ds(j * 16, 16)] = plsc.load_gather(
            e1rows_v, [rowi + j * 16, lanes])

      pltpu.sync_copy(w_v, w_hbm.at[pl.ds(off, _SC_CHUNK)])

  return k(embed2, e1pad, idx_flat, idx_hi, idx_lo)


def _mlp_body(v_ref, w_ref, w1v_ref, w1w_ref, b1_ref, g1_ref, be1_ref,
              w2_ref, b2_ref, g2_ref, be2_ref, w3_ref, b3_ref, g3_ref,
              be3_ref, w4_ref, b4_ref, out_ref, h1_acc):
  i = pl.program_id(0)
  hp = jax.lax.Precision.DEFAULT

  def mm(a, b):
    return jnp.dot(a, b, preferred_element_type=jnp.float32, precision=hp)

  h1 = mm(w_ref[...], w1w_ref[...]) + mm(v_ref[...], w1v_ref[...]) + b1_ref[...]
  h1_acc[pl.ds(i * _TC_CHUNK, _TC_CHUNK), :] = h1

  @pl.when(i == _N_TC_CHUNK - 1)
  def _():
    def bn_relu(h, g, be):
      mu = jnp.mean(h, axis=0, keepdims=True)
      var = jnp.mean((h - mu) ** 2, axis=0, keepdims=True)
      return jnp.maximum((h - mu) * lax.rsqrt(var + 1e-5) * g + be, 0.0)

    h = bn_relu(h1_acc[...], g1_ref[...], be1_ref[...])
    h = bn_relu(mm(h, w2_ref[...]) + b2_ref[...], g2_ref[...], be2_ref[...])
    h = bn_relu(mm(h, w3_ref[...]) + b3_ref[...], g3_ref[...], be3_ref[...])
    o = mm(h, w4_ref[...]) + b4_ref[...]
    out_ref[...] = jax.nn.sigmoid(o)


def _mlp(v, w, w1v, w1w, b1, g1, be1, w2, b2, g2, be2, w3, b3, g3, be3,
         w4, b4):
  full = lambda shape: pl.BlockSpec(shape, lambda i: (0, 0))
  return pl.pallas_call(
      _mlp_body,
      grid=(_N_TC_CHUNK,),
      in_specs=[
          pl.BlockSpec((_TC_CHUNK, _D1), lambda i: (i, 0)),
          pl.BlockSpec((_TC_CHUNK, _NF), lambda i: (i, 0)),
          full(w1v.shape), full(w1w.shape), full(b1.shape), full(g1.shape),
          full(be1.shape), full(w2.shape), full(b2.shape), full(g2.shape),
          full(be2.shape), full(w3.shape), full(b3.shape), full(g3.shape),
          full(be3.shape), full(w4.shape), full(b4.shape),
      ],
      out_specs=pl.BlockSpec((_B, 1), lambda i: (0, 0)),
      out_shape=jax.ShapeDtypeStruct((_B, 1), jnp.float32),
      scratch_shapes=[
          pltpu.VMEM((_B, 128), jnp.float32),
      ],
  )(v, w, w1v, w1w, b1, g1, be1, w2, b2, g2, be2, w3, b3, g3, be3, w4, b4)


def kernel(x, embed1, embed2, W1, b1, g1, be1, W2, b2, g2, be2,
           W3, b3, g3, be3, W4, b4):
  idx_flat = (x + jnp.asarray(_OFFSETS)[None, :]).reshape(_NIDX)
  e1pad = jnp.pad(embed1.reshape(_VOCAB),
                  (0, _V16 * 16 - _VOCAB)).reshape(_V16, _ED)
  v_flat, w_flat = _sc_gather(embed2, e1pad, idx_flat,
                              idx_flat >> 4, idx_flat & 15)
  out = _mlp(
      v_flat.reshape(_B, _D1), w_flat.reshape(_B, _NF), W1[_NF:], W1[:_NF],
      b1.reshape(1, -1), g1.reshape(1, -1), be1.reshape(1, -1),
      W2, b2.reshape(1, -1), g2.reshape(1, -1), be2.reshape(1, -1),
      W3, b3.reshape(1, -1), g3.reshape(1, -1), be3.reshape(1, -1),
      W4, b4.reshape(1, -1),
  )
  return out.reshape(_B)
